# baseline (speedup 1.0000x reference)
import jax
import jax.numpy as jnp
from jax import lax
from jax.experimental import pallas as pl
from jax.experimental.pallas import tpu as pltpu
from jax.experimental.pallas import tpu_sc as plsc

_K = 64
_FACTOR = 6.26
_NC, _NS, _L = 2, 16, 16
_ROWS, _COLS = 128, 8192
_RPW = _ROWS // (_NC * _NS)
_NV = _COLS // _L
_U = 8
_CU = 4
_B30 = 0x40000000


def _f32(bits):
    return plsc.bitcast(bits, jnp.float32)


def _i32(vals):
    return plsc.bitcast(vals, jnp.int32)


def _popcnt(mask):
    return plsc.all_reduce_population_count(mask)


def _select(ck, n0, sum_all, cnt0, s_hi0):
    lanes = lax.iota(jnp.int32, _L)

    def count_pass(bit, n):
        nv = ((n + _L * _U - 1) // (_L * _U)) * _U

        def body(i, carry):
            cnt_v, s_v = carry
            v = ck[pl.ds(i * _L, _L)]
            valid = (i * _L + lanes) < n
            hi = valid & (((v >> bit) & 1) == 1)
            cnt_v = cnt_v + jnp.where(hi, 1, 0)
            s_v = s_v + jnp.where(hi, _f32(v), 0.0)
            return cnt_v, s_v

        cnt_v, s_v = plsc.parallel_loop(
            0, nv, 1, unroll=_U,
            carry=(jnp.zeros((_L,), jnp.int32), jnp.zeros((_L,), jnp.float32))
        )(body)
        return jnp.sum(cnt_v), jnp.sum(s_v)

    def compact_pass(bit, n, want_hi):
        ng = (n + _L * _CU - 1) // (_L * _CU)

        def body(g, wp_v):
            vs = [ck[pl.ds((g * _CU + j) * _L, _L)] for j in range(_CU)]
            for j in range(_CU):
                valid = ((g * _CU + j) * _L + lanes) < n
                hi = ((vs[j] >> bit) & 1) == 1
                keep = valid & (hi == want_hi)
                pos = wp_v + plsc.cumsum(keep.astype(jnp.int32)) - 1
                plsc.store_scatter(ck, [pos], vs[j], mask=keep)
                wp_v = wp_v + _popcnt(keep)
            return wp_v

        wp_v = lax.fori_loop(0, ng, body, jnp.zeros((_L,), jnp.int32))
        return jnp.max(wp_v)

    def cond(state):
        bit, k, n, t, s_sel, s_cand, cnt, s_hi = state
        return (bit >= 0) & (k < n) & (n > _L)

    def step(state):
        bit, k, n, t, s_sel, s_cand, cnt, s_hi = state
        take_hi = cnt >= k
        new_n = compact_pass(bit, n, take_hi)
        t = jnp.where(take_hi, t | (1 << bit), t)
        k = jnp.where(take_hi, k, k - cnt)
        s_sel = jnp.where(take_hi, s_sel, s_sel + s_hi)
        s_cand = jnp.where(take_hi, s_hi, s_cand - s_hi)
        cnt2, s_hi2 = count_pass(jnp.maximum(bit - 1, 0), new_n)
        return bit - 1, k, new_n, t, s_sel, s_cand, cnt2, s_hi2

    init = (jnp.int32(30), jnp.int32(_K), n0, jnp.int32(0),
            jnp.float32(0.0), sum_all, cnt0, s_hi0)
    bit, k, n, t, s_sel, s_cand, _, _ = lax.while_loop(cond, step, init)

    v = ck[pl.ds(0, _L)]
    vf = _f32(v)
    k_v = jnp.full((_L,), k, jnp.int32)
    t_v = jnp.full((_L,), t, jnp.int32)

    def eg_step(i, st):
        k2_v, t2_v, valid = st
        b = bit - i
        hi = valid & (((v >> b) & 1) == 1)
        cnt_v = _popcnt(hi)
        take_hi = cnt_v >= k2_v
        t2_v = jnp.where(take_hi, t2_v | (1 << b), t2_v)
        k2_v = jnp.where(take_hi, k2_v, k2_v - cnt_v)
        valid = valid & (hi == take_hi)
        return k2_v, t2_v, valid

    active = (k < n)
    valid0 = lanes < n
    k2_v, t2_v, _ = lax.fori_loop(
        0, jnp.where(active, bit + 1, 0), eg_step,
        (k_v, t_v, valid0))

    s_gt = jnp.sum(jnp.where(valid0 & (v > t2_v), vf, 0.0))
    val_t = jnp.sum(jnp.where(lanes == 0, _f32(t2_v), 0.0))
    k2 = jnp.max(k2_v)
    t2 = jnp.max(t2_v)
    fin = jnp.where(active, s_sel + s_gt + k2.astype(jnp.float32) * val_t,
                    s_sel + s_cand)
    t = jnp.where(active, t2, t)
    k = jnp.where(active, k2, k)
    return t, k, fin


def _row_compute(xrow, orow, ckp, ckn):
    mask7f = jnp.full((_L,), 0x7FFFFFFF, jnp.int32)

    def p0(i, carry):
        sp_v, sn_v, cp_v, cn_v, shp_v, shn_v = carry
        v = xrow[pl.ds(i * _L, _L)]
        p = jnp.maximum(v, 0.0)
        nn = jnp.maximum(-v, 0.0)
        pk = _i32(p) & mask7f
        nk = _i32(nn) & mask7f
        ckp[pl.ds(i * _L, _L)] = pk
        ckn[pl.ds(i * _L, _L)] = nk
        hip = pk >= _B30
        hin = nk >= _B30
        return (sp_v + p, sn_v + nn,
                cp_v + jnp.where(hip, 1, 0), cn_v + jnp.where(hin, 1, 0),
                shp_v + jnp.where(hip, p, 0.0),
                shn_v + jnp.where(hin, nn, 0.0))

    z_i = jnp.zeros((_L,), jnp.int32)
    z_f = jnp.zeros((_L,), jnp.float32)
    sp_v, sn_v, cp_v, cn_v, shp_v, shn_v = plsc.parallel_loop(
        0, _NV, 1, unroll=_U,
        carry=(z_f, z_f, z_i, z_i, z_f, z_f))(p0)
    sum_p, sum_n = jnp.sum(sp_v), jnp.sum(sn_v)

    tp, krp, ssp = _select(ckp, jnp.int32(_COLS), sum_p,
                           jnp.sum(cp_v), jnp.sum(shp_v))
    tn, krn, ssn = _select(ckn, jnp.int32(_COLS), sum_n,
                           jnp.sum(cn_v), jnp.sum(shn_v))

    p_tmp = _FACTOR * (sum_p - ssp)
    n_tmp = _FACTOR * (sum_n - ssn)

    def out_body(i, carry):
        tcp_v, tcn_v = carry
        v = xrow[pl.ds(i * _L, _L)]
        p = jnp.maximum(v, 0.0)
        nn = jnp.maximum(-v, 0.0)
        pk = _i32(p) & mask7f
        nk = _i32(nn) & mask7f

        tie_p = pk == tp
        rank_p = tcp_v + plsc.cumsum(tie_p.astype(jnp.int32))
        sel_p = (pk > tp) | (tie_p & (rank_p <= krp))
        tcp_v = tcp_v + _popcnt(tie_p)

        tie_n = nk == tn
        rank_n = tcn_v + plsc.cumsum(tie_n.astype(jnp.int32))
        sel_n = (nk > tn) | (tie_n & (rank_n <= krn))
        tcn_v = tcn_v + _popcnt(tie_n)

        out = (jnp.where(sel_p, p + p_tmp, 0.0)
               - jnp.where(sel_n, nn + n_tmp, 0.0))
        orow[pl.ds(i * _L, _L)] = out
        return tcp_v, tcn_v

    plsc.parallel_loop(
        0, _NV, 1, unroll=_U,
        carry=(jnp.zeros((_L,), jnp.int32), jnp.zeros((_L,), jnp.int32))
    )(out_body)


def _sc_body(rpw, x_hbm, o_hbm, xrow, orow, ckp, ckn):
    wid = lax.axis_index("s") * _NC + lax.axis_index("c")

    def per_row(r, carry):
        row = wid * rpw + r
        pltpu.sync_copy(x_hbm.at[row], xrow)
        _row_compute(xrow, orow, ckp, ckn)
        pltpu.sync_copy(orow, o_hbm.at[row])
        return carry

    lax.fori_loop(0, rpw, per_row, jnp.int32(0))


def sc_kernel(x):
    rows = x.shape[0]
    rpw = rows // (_NC * _NS)
    mesh = plsc.VectorSubcoreMesh(core_axis_name="c", subcore_axis_name="s",
                                  num_cores=_NC, num_subcores=_NS)
    f = pl.kernel(
        lambda *a: _sc_body(rpw, *a),
        out_type=jax.ShapeDtypeStruct((rows, _COLS), jnp.float32),
        mesh=mesh,
        compiler_params=pltpu.CompilerParams(needs_layout_passes=False),
        scratch_types=[
            pltpu.VMEM((_COLS,), jnp.float32),
            pltpu.VMEM((_COLS,), jnp.float32),
            pltpu.VMEM((_COLS + _U * _L,), jnp.int32),
            pltpu.VMEM((_COLS + _U * _L,), jnp.int32),
        ],
    )
    return f(x)


def _tc_search_ge(keys, valid, k, nbits, rows):
    def tc_step(i, t):
        cand = t | (1 << (nbits - 1 - i))
        hit = valid & (keys >= cand)
        cnt = jnp.sum(hit.astype(jnp.int32), axis=1, keepdims=True)
        return jnp.where(cnt >= k, cand, t)
    return jax.lax.fori_loop(0, nbits, tc_step,
                             jnp.zeros((rows, 1), jnp.int32))


def _tc_search31_i16(bits, rows):
    hi = (bits >> 16).astype(jnp.int16)
    lo = (bits ^ jnp.int32(0x8000)).astype(jnp.int16)

    def step_a(i, t):
        cand = t | (1 << (14 - i))
        hit = hi >= cand.astype(jnp.int16)
        cnt = jnp.sum(hit.astype(jnp.int32), axis=1, keepdims=True)
        return jnp.where(cnt >= _K, cand, t)

    t_hi = jax.lax.fori_loop(0, 15, step_a, jnp.zeros((rows, 1), jnp.int32))
    eq = hi == t_hi.astype(jnp.int16)
    cnt_gt = jnp.sum((hi > t_hi.astype(jnp.int16)).astype(jnp.int32),
                     axis=1, keepdims=True)

    def step_b(i, t):
        cand = t | (1 << (15 - i))
        cand_b = (cand ^ jnp.int32(0x8000)).astype(jnp.int16)
        hit = eq & (lo >= cand_b)
        cnt = cnt_gt + jnp.sum(hit.astype(jnp.int32), axis=1, keepdims=True)
        return jnp.where(cnt >= _K, cand, t)

    t_lo = jax.lax.fori_loop(0, 16, step_b, jnp.zeros((rows, 1), jnp.int32))
    return (t_hi << 16) | t_lo


def _tc_side_mask(bits, rcol, rows):
    t = _tc_search31_i16(bits, rows)
    gt = bits > t
    cnt_gt = jnp.sum(gt.astype(jnp.int32), axis=1, keepdims=True)
    needed = _K - cnt_gt
    tie = bits == t
    tie_cnt = jnp.sum(tie.astype(jnp.int32), axis=1, keepdims=True)

    t2 = lax.cond(
        jnp.all(tie_cnt == needed),
        lambda: jnp.zeros((rows, 1), jnp.int32),
        lambda: _tc_search_ge(rcol, tie, needed, 13, rows))
    return gt | (tie & (rcol >= t2))


def _tc_body(x_ref, o_ref):
    x = x_ref[...]
    p = jnp.maximum(x, 0.0)
    n = jnp.maximum(-x, 0.0)
    pb = jax.lax.bitcast_convert_type(p, jnp.int32)
    nb = jax.lax.bitcast_convert_type(n, jnp.int32)
    rows, cols = x.shape
    rcol = jax.lax.broadcasted_iota(jnp.int32, (rows, cols), 1)
    rcol = (cols - 1) - rcol

    mp = _tc_side_mask(pb, rcol, rows)
    mn = _tc_side_mask(nb, rcol, rows)
    p_tmp = _FACTOR * jnp.sum(jnp.where(mp, 0.0, p), axis=1, keepdims=True)
    n_tmp = _FACTOR * jnp.sum(jnp.where(mn, 0.0, n), axis=1, keepdims=True)
    o_ref[...] = (jnp.where(mp, p + p_tmp, 0.0)
                  - jnp.where(mn, n + n_tmp, 0.0))


def tc_kernel(x):
    rows, cols = x.shape
    blk = 16
    return pl.pallas_call(
        _tc_body,
        grid=(rows // blk,),
        in_specs=[pl.BlockSpec((blk, cols), lambda i: (i, 0))],
        out_specs=pl.BlockSpec((blk, cols), lambda i: (i, 0)),
        out_shape=jax.ShapeDtypeStruct((rows, cols), x.dtype),
    )(x)


_SC_ROWS = 64


def kernel(x):
    out_sc = sc_kernel(x[:_SC_ROWS])
    out_tc = tc_kernel(x[_SC_ROWS:])
    return jnp.concatenate([out_sc, out_tc], axis=0)


# speedup vs baseline: 1.2186x; 1.2186x over previous
import jax
import jax.numpy as jnp
from jax import lax
from jax.experimental import pallas as pl
from jax.experimental.pallas import tpu as pltpu
from jax.experimental.pallas import tpu_sc as plsc

_K = 64
_FACTOR = 6.26
_NC, _NS, _L = 2, 16, 16
_ROWS, _COLS = 128, 8192
_RPW = _ROWS // (_NC * _NS)
_NV = _COLS // _L
_U = 8
_CU = 4
_B30 = 0x40000000


def _f32(bits):
    return plsc.bitcast(bits, jnp.float32)


def _i32(vals):
    return plsc.bitcast(vals, jnp.int32)


def _popcnt(mask):
    return plsc.all_reduce_population_count(mask)


def _select(ck, n0, sum_all, cnt0, s_hi0):
    lanes = lax.iota(jnp.int32, _L)

    def count_pass(bit, n):
        nv = ((n + _L * _U - 1) // (_L * _U)) * _U

        def body(i, carry):
            cnt_v, s_v = carry
            v = ck[pl.ds(i * _L, _L)]
            valid = (i * _L + lanes) < n
            hi = valid & (((v >> bit) & 1) == 1)
            cnt_v = cnt_v + jnp.where(hi, 1, 0)
            s_v = s_v + jnp.where(hi, _f32(v), 0.0)
            return cnt_v, s_v

        cnt_v, s_v = plsc.parallel_loop(
            0, nv, 1, unroll=_U,
            carry=(jnp.zeros((_L,), jnp.int32), jnp.zeros((_L,), jnp.float32))
        )(body)
        return jnp.sum(cnt_v), jnp.sum(s_v)

    def compact_pass(bit, n, want_hi):
        ng = (n + _L * _CU - 1) // (_L * _CU)

        def body(g, wp_v):
            vs = [ck[pl.ds((g * _CU + j) * _L, _L)] for j in range(_CU)]
            for j in range(_CU):
                valid = ((g * _CU + j) * _L + lanes) < n
                hi = ((vs[j] >> bit) & 1) == 1
                keep = valid & (hi == want_hi)
                pos = wp_v + plsc.cumsum(keep.astype(jnp.int32)) - 1
                plsc.store_scatter(ck, [pos], vs[j], mask=keep)
                wp_v = wp_v + _popcnt(keep)
            return wp_v

        wp_v = lax.fori_loop(0, ng, body, jnp.zeros((_L,), jnp.int32))
        return jnp.max(wp_v)

    def cond(state):
        bit, k, n, t, s_sel, s_cand, cnt, s_hi = state
        return (bit >= 0) & (k < n) & (n > _L)

    def step(state):
        bit, k, n, t, s_sel, s_cand, cnt, s_hi = state
        take_hi = cnt >= k
        new_n = compact_pass(bit, n, take_hi)
        t = jnp.where(take_hi, t | (1 << bit), t)
        k = jnp.where(take_hi, k, k - cnt)
        s_sel = jnp.where(take_hi, s_sel, s_sel + s_hi)
        s_cand = jnp.where(take_hi, s_hi, s_cand - s_hi)
        cnt2, s_hi2 = count_pass(jnp.maximum(bit - 1, 0), new_n)
        return bit - 1, k, new_n, t, s_sel, s_cand, cnt2, s_hi2

    init = (jnp.int32(30), jnp.int32(_K), n0, jnp.int32(0),
            jnp.float32(0.0), sum_all, cnt0, s_hi0)
    bit, k, n, t, s_sel, s_cand, _, _ = lax.while_loop(cond, step, init)

    v = ck[pl.ds(0, _L)]
    vf = _f32(v)
    k_v = jnp.full((_L,), k, jnp.int32)
    t_v = jnp.full((_L,), t, jnp.int32)

    def eg_step(i, st):
        k2_v, t2_v, valid = st
        b = bit - i
        hi = valid & (((v >> b) & 1) == 1)
        cnt_v = _popcnt(hi)
        take_hi = cnt_v >= k2_v
        t2_v = jnp.where(take_hi, t2_v | (1 << b), t2_v)
        k2_v = jnp.where(take_hi, k2_v, k2_v - cnt_v)
        valid = valid & (hi == take_hi)
        return k2_v, t2_v, valid

    active = (k < n)
    valid0 = lanes < n
    k2_v, t2_v, _ = lax.fori_loop(
        0, jnp.where(active, bit + 1, 0), eg_step,
        (k_v, t_v, valid0))

    s_gt = jnp.sum(jnp.where(valid0 & (v > t2_v), vf, 0.0))
    val_t = jnp.sum(jnp.where(lanes == 0, _f32(t2_v), 0.0))
    k2 = jnp.max(k2_v)
    t2 = jnp.max(t2_v)
    fin = jnp.where(active, s_sel + s_gt + k2.astype(jnp.float32) * val_t,
                    s_sel + s_cand)
    cnt_ties = jnp.max(_popcnt(valid0 & (v == t2_v)))
    no_surplus = (~active) | ((bit >= 0) & (cnt_ties == k2))
    t = jnp.where(active, t2, t)
    k = jnp.where(active, k2, k)
    return t, k, fin, no_surplus


def _row_compute(xrow, orow, ckp, ckn):
    mask7f = jnp.full((_L,), 0x7FFFFFFF, jnp.int32)

    def p0(i, carry):
        sp_v, sn_v, cp_v, cn_v, shp_v, shn_v = carry
        v = xrow[pl.ds(i * _L, _L)]
        p = jnp.maximum(v, 0.0)
        nn = jnp.maximum(-v, 0.0)
        pk = _i32(p) & mask7f
        nk = _i32(nn) & mask7f
        ckp[pl.ds(i * _L, _L)] = pk
        ckn[pl.ds(i * _L, _L)] = nk
        hip = pk >= _B30
        hin = nk >= _B30
        return (sp_v + p, sn_v + nn,
                cp_v + jnp.where(hip, 1, 0), cn_v + jnp.where(hin, 1, 0),
                shp_v + jnp.where(hip, p, 0.0),
                shn_v + jnp.where(hin, nn, 0.0))

    z_i = jnp.zeros((_L,), jnp.int32)
    z_f = jnp.zeros((_L,), jnp.float32)
    sp_v, sn_v, cp_v, cn_v, shp_v, shn_v = plsc.parallel_loop(
        0, _NV, 1, unroll=_U,
        carry=(z_f, z_f, z_i, z_i, z_f, z_f))(p0)
    sum_p, sum_n = jnp.sum(sp_v), jnp.sum(sn_v)

    tp, krp, ssp, okp = _select(ckp, jnp.int32(_COLS), sum_p,
                                jnp.sum(cp_v), jnp.sum(shp_v))
    tn, krn, ssn, okn = _select(ckn, jnp.int32(_COLS), sum_n,
                                jnp.sum(cn_v), jnp.sum(shn_v))

    p_tmp = _FACTOR * (sum_p - ssp)
    n_tmp = _FACTOR * (sum_n - ssn)
    vtp = jnp.sum(jnp.where(lax.iota(jnp.int32, _L) == 0,
                            _f32(jnp.full((_L,), tp, jnp.int32)), 0.0))
    vtn = jnp.sum(jnp.where(lax.iota(jnp.int32, _L) == 0,
                            _f32(jnp.full((_L,), tn, jnp.int32)), 0.0))

    def fast_out():
        def body(i, carry):
            v = xrow[pl.ds(i * _L, _L)]
            p = jnp.maximum(v, 0.0)
            nn = jnp.maximum(-v, 0.0)
            out = (jnp.where(p >= vtp, p + p_tmp, 0.0)
                   - jnp.where(nn >= vtn, nn + n_tmp, 0.0))
            orow[pl.ds(i * _L, _L)] = out
            return carry

        plsc.parallel_loop(0, _NV, 1, unroll=_U,
                           carry=jnp.int32(0))(body)

    def slow_out():
        def body(i, carry):
            tcp_v, tcn_v = carry
            v = xrow[pl.ds(i * _L, _L)]
            p = jnp.maximum(v, 0.0)
            nn = jnp.maximum(-v, 0.0)
            pk = _i32(p) & mask7f
            nk = _i32(nn) & mask7f

            tie_p = pk == tp
            rank_p = tcp_v + plsc.cumsum(tie_p.astype(jnp.int32))
            sel_p = (pk > tp) | (tie_p & (rank_p <= krp))
            tcp_v = tcp_v + _popcnt(tie_p)

            tie_n = nk == tn
            rank_n = tcn_v + plsc.cumsum(tie_n.astype(jnp.int32))
            sel_n = (nk > tn) | (tie_n & (rank_n <= krn))
            tcn_v = tcn_v + _popcnt(tie_n)

            out = (jnp.where(sel_p, p + p_tmp, 0.0)
                   - jnp.where(sel_n, nn + n_tmp, 0.0))
            orow[pl.ds(i * _L, _L)] = out
            return tcp_v, tcn_v

        plsc.parallel_loop(
            0, _NV, 1, unroll=_U,
            carry=(jnp.zeros((_L,), jnp.int32), jnp.zeros((_L,), jnp.int32))
        )(body)

    lax.cond(okp & okn, fast_out, slow_out)


def _sc_body(rpw, x_hbm, o_hbm, xrow, orow, ckp, ckn):
    wid = lax.axis_index("s") * _NC + lax.axis_index("c")

    def per_row(r, carry):
        row = wid * rpw + r
        pltpu.sync_copy(x_hbm.at[row], xrow)
        _row_compute(xrow, orow, ckp, ckn)
        pltpu.sync_copy(orow, o_hbm.at[row])
        return carry

    lax.fori_loop(0, rpw, per_row, jnp.int32(0))


def sc_kernel(x):
    rows = x.shape[0]
    rpw = rows // (_NC * _NS)
    mesh = plsc.VectorSubcoreMesh(core_axis_name="c", subcore_axis_name="s",
                                  num_cores=_NC, num_subcores=_NS)
    f = pl.kernel(
        lambda *a: _sc_body(rpw, *a),
        out_type=jax.ShapeDtypeStruct((rows, _COLS), jnp.float32),
        mesh=mesh,
        compiler_params=pltpu.CompilerParams(needs_layout_passes=False),
        scratch_types=[
            pltpu.VMEM((_COLS,), jnp.float32),
            pltpu.VMEM((_COLS,), jnp.float32),
            pltpu.VMEM((_COLS + _U * _L,), jnp.int32),
            pltpu.VMEM((_COLS + _U * _L,), jnp.int32),
        ],
    )
    return f(x)


def _tc_search_ge(keys, valid, k, nbits, rows):
    def tc_step(i, t):
        cand = t | (1 << (nbits - 1 - i))
        hit = valid & (keys >= cand)
        cnt = jnp.sum(hit.astype(jnp.int32), axis=1, keepdims=True)
        return jnp.where(cnt >= k, cand, t)
    return jax.lax.fori_loop(0, nbits, tc_step,
                             jnp.zeros((rows, 1), jnp.int32))


def _tc_side_mask(bits, rcol, rows):
    ones = bits >= 0
    t = _tc_search_ge(bits, ones, _K, 31, rows)
    gt = bits > t
    cnt_gt = jnp.sum(gt.astype(jnp.int32), axis=1, keepdims=True)
    needed = _K - cnt_gt
    tie = bits == t
    tie_cnt = jnp.sum(tie.astype(jnp.int32), axis=1, keepdims=True)

    t2 = lax.cond(
        jnp.all(tie_cnt == needed),
        lambda: jnp.zeros((rows, 1), jnp.int32),
        lambda: _tc_search_ge(rcol, tie, needed, 13, rows))
    return gt | (tie & (rcol >= t2))


def _tc_body(x_ref, o_ref):
    x = x_ref[...]
    p = jnp.maximum(x, 0.0)
    n = jnp.maximum(-x, 0.0)
    pb = jax.lax.bitcast_convert_type(p, jnp.int32)
    nb = jax.lax.bitcast_convert_type(n, jnp.int32)
    rows, cols = x.shape
    rcol = jax.lax.broadcasted_iota(jnp.int32, (rows, cols), 1)
    rcol = (cols - 1) - rcol

    mp = _tc_side_mask(pb, rcol, rows)
    mn = _tc_side_mask(nb, rcol, rows)
    p_tmp = _FACTOR * jnp.sum(jnp.where(mp, 0.0, p), axis=1, keepdims=True)
    n_tmp = _FACTOR * jnp.sum(jnp.where(mn, 0.0, n), axis=1, keepdims=True)
    o_ref[...] = (jnp.where(mp, p + p_tmp, 0.0)
                  - jnp.where(mn, n + n_tmp, 0.0))


def tc_kernel(x):
    rows, cols = x.shape
    blk = 16
    return pl.pallas_call(
        _tc_body,
        grid=(rows // blk,),
        in_specs=[pl.BlockSpec((blk, cols), lambda i: (i, 0))],
        out_specs=pl.BlockSpec((blk, cols), lambda i: (i, 0)),
        out_shape=jax.ShapeDtypeStruct((rows, cols), x.dtype),
    )(x)


_SC_ROWS = 64


def kernel(x):
    out_sc = sc_kernel(x[:_SC_ROWS])
    out_tc = tc_kernel(x[_SC_ROWS:])
    return jnp.concatenate([out_sc, out_tc], axis=0)


# speedup vs baseline: 1.3784x; 1.1311x over previous
import jax
import jax.numpy as jnp
from jax import lax
from jax.experimental import pallas as pl
from jax.experimental.pallas import tpu as pltpu
from jax.experimental.pallas import tpu_sc as plsc

_K = 64
_FACTOR = 6.26
_NC, _NS, _L = 2, 16, 16
_ROWS, _COLS = 128, 8192
_RPW = _ROWS // (_NC * _NS)
_NV = _COLS // _L
_U = 8
_CU = 4
_B30 = 0x40000000


def _f32(bits):
    return plsc.bitcast(bits, jnp.float32)


def _i32(vals):
    return plsc.bitcast(vals, jnp.int32)


def _popcnt(mask):
    return plsc.all_reduce_population_count(mask)


def _select(ck, n0, sum_all, cnt0, s_hi0):
    lanes = lax.iota(jnp.int32, _L)

    def count_pass(bit, n):
        nv = ((n + _L * _U - 1) // (_L * _U)) * _U

        def body(i, carry):
            cnt_v, s_v = carry
            v = ck[pl.ds(i * _L, _L)]
            valid = (i * _L + lanes) < n
            hi = valid & (((v >> bit) & 1) == 1)
            cnt_v = cnt_v + jnp.where(hi, 1, 0)
            s_v = s_v + jnp.where(hi, _f32(v), 0.0)
            return cnt_v, s_v

        cnt_v, s_v = plsc.parallel_loop(
            0, nv, 1, unroll=_U,
            carry=(jnp.zeros((_L,), jnp.int32), jnp.zeros((_L,), jnp.float32))
        )(body)
        return jnp.sum(cnt_v), jnp.sum(s_v)

    def compact_pass(bit, n, want_hi):
        ng = (n + _L * _CU - 1) // (_L * _CU)

        def body(g, wp_v):
            vs = [ck[pl.ds((g * _CU + j) * _L, _L)] for j in range(_CU)]
            for j in range(_CU):
                valid = ((g * _CU + j) * _L + lanes) < n
                hi = ((vs[j] >> bit) & 1) == 1
                keep = valid & (hi == want_hi)
                pos = wp_v + plsc.cumsum(keep.astype(jnp.int32)) - 1
                plsc.store_scatter(ck, [pos], vs[j], mask=keep)
                wp_v = wp_v + _popcnt(keep)
            return wp_v

        wp_v = lax.fori_loop(0, ng, body, jnp.zeros((_L,), jnp.int32))
        return jnp.max(wp_v)

    def cond(state):
        bit, k, n, t, s_sel, s_cand, cnt, s_hi = state
        return (bit >= 0) & (k < n) & (n > _L)

    def step(state):
        bit, k, n, t, s_sel, s_cand, cnt, s_hi = state
        take_hi = cnt >= k
        new_n = compact_pass(bit, n, take_hi)
        t = jnp.where(take_hi, t | (1 << bit), t)
        k = jnp.where(take_hi, k, k - cnt)
        s_sel = jnp.where(take_hi, s_sel, s_sel + s_hi)
        s_cand = jnp.where(take_hi, s_hi, s_cand - s_hi)
        cnt2, s_hi2 = count_pass(jnp.maximum(bit - 1, 0), new_n)
        return bit - 1, k, new_n, t, s_sel, s_cand, cnt2, s_hi2

    init = (jnp.int32(30), jnp.int32(_K), n0, jnp.int32(0),
            jnp.float32(0.0), sum_all, cnt0, s_hi0)
    bit, k, n, t, s_sel, s_cand, _, _ = lax.while_loop(cond, step, init)

    v = ck[pl.ds(0, _L)]
    vf = _f32(v)
    k_v = jnp.full((_L,), k, jnp.int32)
    t_v = jnp.full((_L,), t, jnp.int32)

    def eg_step(i, st):
        k2_v, t2_v, valid = st
        b = bit - i
        hi = valid & (((v >> b) & 1) == 1)
        cnt_v = _popcnt(hi)
        take_hi = cnt_v >= k2_v
        t2_v = jnp.where(take_hi, t2_v | (1 << b), t2_v)
        k2_v = jnp.where(take_hi, k2_v, k2_v - cnt_v)
        valid = valid & (hi == take_hi)
        return k2_v, t2_v, valid

    active = (k < n)
    valid0 = lanes < n
    k2_v, t2_v, _ = lax.fori_loop(
        0, jnp.where(active, bit + 1, 0), eg_step,
        (k_v, t_v, valid0))

    s_gt = jnp.sum(jnp.where(valid0 & (v > t2_v), vf, 0.0))
    val_t = jnp.sum(jnp.where(lanes == 0, _f32(t2_v), 0.0))
    k2 = jnp.max(k2_v)
    t2 = jnp.max(t2_v)
    fin = jnp.where(active, s_sel + s_gt + k2.astype(jnp.float32) * val_t,
                    s_sel + s_cand)
    cnt_ties = jnp.max(_popcnt(valid0 & (v == t2_v)))
    no_surplus = (~active) | ((bit >= 0) & (cnt_ties == k2))
    t = jnp.where(active, t2, t)
    k = jnp.where(active, k2, k)
    return t, k, fin, no_surplus


def _row_compute(xrow, orow, ckp, ckn):
    mask7f = jnp.full((_L,), 0x7FFFFFFF, jnp.int32)

    def p0(i, carry):
        sp_v, sn_v, cp_v, cn_v, shp_v, shn_v = carry
        v = xrow[pl.ds(i * _L, _L)]
        p = jnp.maximum(v, 0.0)
        nn = jnp.maximum(-v, 0.0)
        pk = _i32(p) & mask7f
        nk = _i32(nn) & mask7f
        ckp[pl.ds(i * _L, _L)] = pk
        ckn[pl.ds(i * _L, _L)] = nk
        hip = pk >= _B30
        hin = nk >= _B30
        return (sp_v + p, sn_v + nn,
                cp_v + jnp.where(hip, 1, 0), cn_v + jnp.where(hin, 1, 0),
                shp_v + jnp.where(hip, p, 0.0),
                shn_v + jnp.where(hin, nn, 0.0))

    z_i = jnp.zeros((_L,), jnp.int32)
    z_f = jnp.zeros((_L,), jnp.float32)
    sp_v, sn_v, cp_v, cn_v, shp_v, shn_v = plsc.parallel_loop(
        0, _NV, 1, unroll=_U,
        carry=(z_f, z_f, z_i, z_i, z_f, z_f))(p0)
    sum_p, sum_n = jnp.sum(sp_v), jnp.sum(sn_v)

    tp, krp, ssp, okp = _select(ckp, jnp.int32(_COLS), sum_p,
                                jnp.sum(cp_v), jnp.sum(shp_v))
    tn, krn, ssn, okn = _select(ckn, jnp.int32(_COLS), sum_n,
                                jnp.sum(cn_v), jnp.sum(shn_v))

    p_tmp = _FACTOR * (sum_p - ssp)
    n_tmp = _FACTOR * (sum_n - ssn)
    vtp = jnp.sum(jnp.where(lax.iota(jnp.int32, _L) == 0,
                            _f32(jnp.full((_L,), tp, jnp.int32)), 0.0))
    vtn = jnp.sum(jnp.where(lax.iota(jnp.int32, _L) == 0,
                            _f32(jnp.full((_L,), tn, jnp.int32)), 0.0))

    def fast_out():
        def body(i, carry):
            v = xrow[pl.ds(i * _L, _L)]
            p = jnp.maximum(v, 0.0)
            nn = jnp.maximum(-v, 0.0)
            out = (jnp.where(p >= vtp, p + p_tmp, 0.0)
                   - jnp.where(nn >= vtn, nn + n_tmp, 0.0))
            orow[pl.ds(i * _L, _L)] = out
            return carry

        plsc.parallel_loop(0, _NV, 1, unroll=_U,
                           carry=jnp.int32(0))(body)

    def slow_out():
        def body(i, carry):
            tcp_v, tcn_v = carry
            v = xrow[pl.ds(i * _L, _L)]
            p = jnp.maximum(v, 0.0)
            nn = jnp.maximum(-v, 0.0)
            pk = _i32(p) & mask7f
            nk = _i32(nn) & mask7f

            tie_p = pk == tp
            rank_p = tcp_v + plsc.cumsum(tie_p.astype(jnp.int32))
            sel_p = (pk > tp) | (tie_p & (rank_p <= krp))
            tcp_v = tcp_v + _popcnt(tie_p)

            tie_n = nk == tn
            rank_n = tcn_v + plsc.cumsum(tie_n.astype(jnp.int32))
            sel_n = (nk > tn) | (tie_n & (rank_n <= krn))
            tcn_v = tcn_v + _popcnt(tie_n)

            out = (jnp.where(sel_p, p + p_tmp, 0.0)
                   - jnp.where(sel_n, nn + n_tmp, 0.0))
            orow[pl.ds(i * _L, _L)] = out
            return tcp_v, tcn_v

        plsc.parallel_loop(
            0, _NV, 1, unroll=_U,
            carry=(jnp.zeros((_L,), jnp.int32), jnp.zeros((_L,), jnp.int32))
        )(body)

    lax.cond(okp & okn, fast_out, slow_out)


def _sc_body(rpw, x_hbm, o_hbm, xrow, orow, ckp, ckn):
    wid = lax.axis_index("s") * _NC + lax.axis_index("c")

    def per_row(r, carry):
        row = wid * rpw + r
        pltpu.sync_copy(x_hbm.at[row], xrow)
        _row_compute(xrow, orow, ckp, ckn)
        pltpu.sync_copy(orow, o_hbm.at[row])
        return carry

    lax.fori_loop(0, rpw, per_row, jnp.int32(0))


def sc_kernel(x):
    rows = x.shape[0]
    rpw = rows // (_NC * _NS)
    mesh = plsc.VectorSubcoreMesh(core_axis_name="c", subcore_axis_name="s",
                                  num_cores=_NC, num_subcores=_NS)
    f = pl.kernel(
        lambda *a: _sc_body(rpw, *a),
        out_type=jax.ShapeDtypeStruct((rows, _COLS), jnp.float32),
        mesh=mesh,
        compiler_params=pltpu.CompilerParams(needs_layout_passes=False),
        scratch_types=[
            pltpu.VMEM((_COLS,), jnp.float32),
            pltpu.VMEM((_COLS,), jnp.float32),
            pltpu.VMEM((_COLS + _U * _L,), jnp.int32),
            pltpu.VMEM((_COLS + _U * _L,), jnp.int32),
        ],
    )
    return f(x)


def _tc_search_ge(keys, valid, k, nbits, rows):
    def tc_step(i, t):
        cand = t | (1 << (nbits - 1 - i))
        hit = valid & (keys >= cand)
        cnt = jnp.sum(hit.astype(jnp.int32), axis=1, keepdims=True)
        return jnp.where(cnt >= k, cand, t)
    return jax.lax.fori_loop(0, nbits, tc_step,
                             jnp.zeros((rows, 1), jnp.int32))


def _tc_side_mask(bits, rcol, rows):
    ones = bits >= 0
    t = _tc_search_ge(bits, ones, _K, 31, rows)
    gt = bits > t
    cnt_gt = jnp.sum(gt.astype(jnp.int32), axis=1, keepdims=True)
    needed = _K - cnt_gt
    tie = bits == t
    tie_cnt = jnp.sum(tie.astype(jnp.int32), axis=1, keepdims=True)

    t2 = lax.cond(
        jnp.all(tie_cnt == needed),
        lambda: jnp.zeros((rows, 1), jnp.int32),
        lambda: _tc_search_ge(rcol, tie, needed, 13, rows))
    return gt | (tie & (rcol >= t2))


def _tc_body(x_ref, o_ref):
    x = x_ref[...]
    p = jnp.maximum(x, 0.0)
    n = jnp.maximum(-x, 0.0)
    pb = jax.lax.bitcast_convert_type(p, jnp.int32)
    nb = jax.lax.bitcast_convert_type(n, jnp.int32)
    rows, cols = x.shape
    rcol = jax.lax.broadcasted_iota(jnp.int32, (rows, cols), 1)
    rcol = (cols - 1) - rcol

    mp = _tc_side_mask(pb, rcol, rows)
    mn = _tc_side_mask(nb, rcol, rows)
    p_tmp = _FACTOR * jnp.sum(jnp.where(mp, 0.0, p), axis=1, keepdims=True)
    n_tmp = _FACTOR * jnp.sum(jnp.where(mn, 0.0, n), axis=1, keepdims=True)
    o_ref[...] = (jnp.where(mp, p + p_tmp, 0.0)
                  - jnp.where(mn, n + n_tmp, 0.0))


def tc_kernel(x):
    rows, cols = x.shape
    blk = 16
    return pl.pallas_call(
        _tc_body,
        grid=(rows // blk,),
        in_specs=[pl.BlockSpec((blk, cols), lambda i: (i, 0))],
        out_specs=pl.BlockSpec((blk, cols), lambda i: (i, 0)),
        out_shape=jax.ShapeDtypeStruct((rows, cols), x.dtype),
    )(x)


_SC_ROWS = 96


def kernel(x):
    out_sc = sc_kernel(x[:_SC_ROWS])
    out_tc = tc_kernel(x[_SC_ROWS:])
    return jnp.concatenate([out_sc, out_tc], axis=0)
